# Initial kernel scaffold; baseline (speedup 1.0000x reference)
import jax
import jax.numpy as jnp
from jax.experimental import pallas as pl


def kernel(inputs, index, lam):
    raise NotImplementedError("write your pallas kernel here")


# speedup vs baseline: 1.0154x; 1.0154x over previous
import jax
import jax.numpy as jnp
from jax.experimental import pallas as pl
from jax.experimental.pallas import tpu as pltpu


def _mix_body(idx_ref, x_ref, g_ref, lam_ref, o_ref):
    lam = lam_ref[0]
    o_ref[...] = lam * x_ref[...] + (1.0 - lam) * g_ref[...]


def kernel(inputs, index, lam):
    B = inputs.shape[0]
    D = inputs.shape[1] * inputs.shape[2] * inputs.shape[3]
    x = inputs.reshape(B, D // 128, 128)
    idx = index.astype(jnp.int32)
    lam_arr = jnp.asarray(lam, jnp.float32).reshape(1)
    blk = (1, D // 128, 128)
    out = pl.pallas_call(
        _mix_body,
        grid_spec=pltpu.PrefetchScalarGridSpec(
            num_scalar_prefetch=1,
            grid=(B,),
            in_specs=[
                pl.BlockSpec(blk, lambda i, idx_ref: (i, 0, 0)),
                pl.BlockSpec(blk, lambda i, idx_ref: (idx_ref[i], 0, 0)),
                pl.BlockSpec(memory_space=pltpu.SMEM),
            ],
            out_specs=pl.BlockSpec(blk, lambda i, idx_ref: (i, 0, 0)),
        ),
        out_shape=jax.ShapeDtypeStruct(x.shape, x.dtype),
    )(idx, x, x, lam_arr)
    return out.reshape(inputs.shape)
